# baseline (speedup 1.0000x reference)
import functools

import jax
import jax.numpy as jnp
from jax.experimental import pallas as pl
from jax.experimental.pallas import tpu as pltpu

ROWS = 8


def _body(rows, di_ref, w2s_ref, pk_ref, logits_ref, out_ref, rs_ref, seg_ref):
    i = pl.program_id(0)
    first_block = i == 0

    packed_row = pk_ref[...]
    lanes = jax.lax.broadcasted_iota(jnp.int32, packed_row.shape, 1)
    w2s_row = w2s_ref[...]

    rs = rs_ref[...]
    seg = seg_ref[...]

    tok_blk = di_ref[...]
    widx_blk = jax.lax.shift_right_logical(tok_blk, 3)
    psel_blk = jnp.sum(
        jnp.where(lanes[None] == widx_blk, packed_row[None], 0),
        axis=2, keepdims=True)
    syl_blk = jax.lax.shift_right_logical(
        psel_blk, (tok_blk & 7) * 4) & 15

    for r in range(rows):
        tok = tok_blk[r]
        is_sep = tok == 7
        sep_i = is_sep.astype(jnp.int32)
        syl = syl_blk[r]

        if r == 0:
            seg = jnp.where(first_block, sep_i,
                            jnp.minimum(seg + sep_i, 5))
        else:
            seg = jnp.minimum(seg + sep_i, 5)
        rs_dec = jnp.maximum(rs - syl, 0)
        pat = jnp.where(seg == 5, 0, jnp.where((seg == 0) | (seg == 2), 5, 7))
        if r == 0:
            sep_val = jnp.where(first_block, 7, pat)
            rs = jnp.where(is_sep, sep_val,
                           jnp.where(first_block, 5, rs_dec))
        else:
            rs = jnp.where(is_sep, pat, rs_dec)

        out_ref[r] = jnp.where(w2s_row > rs, -jnp.inf, logits_ref[r])

    rs_ref[...] = rs
    seg_ref[...] = seg


def kernel(logits, decoder_input, word2syllables):
    seq, batch = decoder_input.shape
    vocab = logits.shape[-1]

    di3 = decoder_input.reshape(seq, batch, 1)
    w2s2 = word2syllables.reshape(1, vocab)
    npk = (vocab + 7) // 8
    w2s_grp = jnp.zeros((npk * 8,), jnp.int32).at[:vocab].set(
        word2syllables.astype(jnp.int32) & 15).reshape(npk, 8)
    shifts = (jnp.arange(8, dtype=jnp.int32) * 4)[None, :]
    packed = jnp.sum(w2s_grp << shifts, axis=1).astype(jnp.int32)
    npk_pad = (-npk) % 128
    packed_padded = jnp.concatenate(
        [packed, jnp.zeros((npk_pad,), jnp.int32)]).reshape(1, npk + npk_pad)

    out = pl.pallas_call(
        functools.partial(_body, ROWS),
        grid=(seq // ROWS,),
        in_specs=[
            pl.BlockSpec((ROWS, batch, 1), lambda i: (i, 0, 0)),
            pl.BlockSpec((1, vocab), lambda i: (0, 0)),
            pl.BlockSpec((1, npk + npk_pad), lambda i: (0, 0)),
            pl.BlockSpec((ROWS, batch, vocab), lambda i: (i, 0, 0)),
        ],
        out_specs=pl.BlockSpec((ROWS, batch, vocab), lambda i: (i, 0, 0)),
        out_shape=jax.ShapeDtypeStruct((seq, batch, vocab), jnp.float32),
        scratch_shapes=[
            pltpu.VMEM((batch, 1), jnp.int32),
            pltpu.VMEM((batch, 1), jnp.int32),
        ],
    )(di3, w2s2, packed_padded, logits)
    return out


# speedup vs baseline: 1.0451x; 1.0451x over previous
import functools

import jax
import jax.numpy as jnp
from jax.experimental import pallas as pl
from jax.experimental.pallas import tpu as pltpu

ROWS = 16


def _body(rows, di_ref, w2s_ref, pk_ref, logits_ref, out_ref, rs_ref, seg_ref):
    i = pl.program_id(0)
    first_block = i == 0

    packed_row = pk_ref[...]
    lanes = jax.lax.broadcasted_iota(jnp.int32, packed_row.shape, 1)
    w2s_row = w2s_ref[...]

    rs = rs_ref[...]
    seg = seg_ref[...]

    tok_blk = di_ref[...]
    widx_blk = jax.lax.shift_right_logical(tok_blk, 3)
    psel_blk = jnp.sum(
        jnp.where(lanes[None] == widx_blk, packed_row[None], 0),
        axis=2, keepdims=True)
    syl_blk = jax.lax.shift_right_logical(
        psel_blk, (tok_blk & 7) * 4) & 15

    for r in range(rows):
        tok = tok_blk[r]
        is_sep = tok == 7
        sep_i = is_sep.astype(jnp.int32)
        syl = syl_blk[r]

        if r == 0:
            seg = jnp.where(first_block, sep_i,
                            jnp.minimum(seg + sep_i, 5))
        else:
            seg = jnp.minimum(seg + sep_i, 5)
        rs_dec = jnp.maximum(rs - syl, 0)
        pat = jnp.where(seg == 5, 0, jnp.where((seg == 0) | (seg == 2), 5, 7))
        if r == 0:
            sep_val = jnp.where(first_block, 7, pat)
            rs = jnp.where(is_sep, sep_val,
                           jnp.where(first_block, 5, rs_dec))
        else:
            rs = jnp.where(is_sep, pat, rs_dec)

        out_ref[r] = jnp.where(w2s_row > rs, -jnp.inf, logits_ref[r])

    rs_ref[...] = rs
    seg_ref[...] = seg


def kernel(logits, decoder_input, word2syllables):
    seq, batch = decoder_input.shape
    vocab = logits.shape[-1]

    di3 = decoder_input.reshape(seq, batch, 1)
    w2s2 = word2syllables.reshape(1, vocab)
    npk = (vocab + 7) // 8
    w2s_grp = jnp.zeros((npk * 8,), jnp.int32).at[:vocab].set(
        word2syllables.astype(jnp.int32) & 15).reshape(npk, 8)
    shifts = (jnp.arange(8, dtype=jnp.int32) * 4)[None, :]
    packed = jnp.sum(w2s_grp << shifts, axis=1).astype(jnp.int32)
    npk_pad = (-npk) % 128
    packed_padded = jnp.concatenate(
        [packed, jnp.zeros((npk_pad,), jnp.int32)]).reshape(1, npk + npk_pad)

    out = pl.pallas_call(
        functools.partial(_body, ROWS),
        grid=(seq // ROWS,),
        in_specs=[
            pl.BlockSpec((ROWS, batch, 1), lambda i: (i, 0, 0)),
            pl.BlockSpec((1, vocab), lambda i: (0, 0)),
            pl.BlockSpec((1, npk + npk_pad), lambda i: (0, 0)),
            pl.BlockSpec((ROWS, batch, vocab), lambda i: (i, 0, 0)),
        ],
        out_specs=pl.BlockSpec((ROWS, batch, vocab), lambda i: (i, 0, 0)),
        out_shape=jax.ShapeDtypeStruct((seq, batch, vocab), jnp.float32),
        scratch_shapes=[
            pltpu.VMEM((batch, 1), jnp.int32),
            pltpu.VMEM((batch, 1), jnp.int32),
        ],
    )(di3, w2s2, packed_padded, logits)
    return out
